# baseline (speedup 1.0000x reference)
import functools

import jax
import jax.numpy as jnp
from jax import lax
from jax.experimental import pallas as pl
from jax.experimental.pallas import tpu as pltpu
from jax.experimental.pallas import tpu_sc as plsc

_CHUNK = 128


@functools.cache
def _build(B, D, dtype):
    info = plsc.get_sparse_core_info()
    NC, NS = info.num_cores, info.num_subcores
    NW = NC * NS
    b_per_w = B // NW
    n_chunks = b_per_w // _CHUNK
    mesh = plsc.VectorSubcoreMesh(core_axis_name="c", subcore_axis_name="s")

    @functools.partial(
        pl.kernel,
        mesh=mesh,
        compiler_params=pltpu.CompilerParams(use_tc_tiling_on_sc=False),
        out_type=(
            jax.ShapeDtypeStruct((B, D), dtype),
            jax.ShapeDtypeStruct((B, D), dtype),
        ),
        scratch_types=[
            pltpu.VMEM((b_per_w,), jnp.int32),
            pltpu.VMEM((b_per_w, D), dtype),
            pltpu.VMEM((b_per_w,), jnp.int32),
            pltpu.VMEM((b_per_w, D), dtype),
            pltpu.SemaphoreType.DMA,
            pltpu.SemaphoreType.DMA,
        ],
    )
    def k(users_hbm, items_hbm, utab_hbm, itab_hbm, uout_hbm, iout_hbm,
          uidx_v, urows_v, iidx_v, irows_v, usem, isem):
        wid = lax.axis_index("s") * NC + lax.axis_index("c")
        base = wid * b_per_w
        pltpu.sync_copy(users_hbm.at[pl.ds(base, b_per_w)], uidx_v)
        pltpu.sync_copy(items_hbm.at[pl.ds(base, b_per_w)], iidx_v)
        ucps = []
        icps = []
        for j in range(n_chunks):
            s = pl.ds(j * _CHUNK, _CHUNK)
            ucps.append(pltpu.async_copy(
                utab_hbm.at[uidx_v.at[s]], urows_v.at[s], usem))
            icps.append(pltpu.async_copy(
                itab_hbm.at[iidx_v.at[s]], irows_v.at[s], isem))
        for cp in ucps:
            cp.wait()
        pltpu.sync_copy(urows_v, uout_hbm.at[pl.ds(base, b_per_w)])
        for cp in icps:
            cp.wait()
        pltpu.sync_copy(irows_v, iout_hbm.at[pl.ds(base, b_per_w)])

    return k


def kernel(users, items, user_table, item_table):
    B = users.shape[0]
    D = user_table.shape[1]
    k = _build(B, D, user_table.dtype)
    out = k(users.astype(jnp.int32), items.astype(jnp.int32),
            user_table, item_table)
    return (out[0], out[1])


# speedup vs baseline: 1.0022x; 1.0022x over previous
import functools

import jax
import jax.numpy as jnp
from jax import lax
from jax.experimental import pallas as pl
from jax.experimental.pallas import tpu as pltpu
from jax.experimental.pallas import tpu_sc as plsc

_CHUNK = 128


@functools.cache
def _build(B, D, dtype):
    info = plsc.get_sparse_core_info()
    NC, NS = info.num_cores, info.num_subcores
    NW = NC * NS
    b_per_w = B // NW
    n_chunks = b_per_w // _CHUNK
    mesh = plsc.VectorSubcoreMesh(core_axis_name="c", subcore_axis_name="s")

    @functools.partial(
        pl.kernel,
        mesh=mesh,
        compiler_params=pltpu.CompilerParams(
            use_tc_tiling_on_sc=False, skip_device_barrier=True),
        out_type=(
            jax.ShapeDtypeStruct((B, D), dtype),
            jax.ShapeDtypeStruct((B, D), dtype),
        ),
        scratch_types=[
            pltpu.VMEM((b_per_w,), jnp.int32),
            pltpu.VMEM((b_per_w, D), dtype),
            pltpu.VMEM((b_per_w,), jnp.int32),
            pltpu.VMEM((b_per_w, D), dtype),
            pltpu.SemaphoreType.DMA,
            pltpu.SemaphoreType.DMA,
        ],
    )
    def k(users_hbm, items_hbm, utab_hbm, itab_hbm, uout_hbm, iout_hbm,
          uidx_v, urows_v, iidx_v, irows_v, usem, isem):
        wid = lax.axis_index("s") * NC + lax.axis_index("c")
        base = wid * b_per_w
        pltpu.sync_copy(users_hbm.at[pl.ds(base, b_per_w)], uidx_v)
        pltpu.sync_copy(items_hbm.at[pl.ds(base, b_per_w)], iidx_v)
        ucps = []
        icps = []
        for j in range(n_chunks):
            s = pl.ds(j * _CHUNK, _CHUNK)
            ucps.append(pltpu.async_copy(
                utab_hbm.at[uidx_v.at[s]], urows_v.at[s], usem))
            icps.append(pltpu.async_copy(
                itab_hbm.at[iidx_v.at[s]], irows_v.at[s], isem))
        for cp in ucps:
            cp.wait()
        pltpu.sync_copy(urows_v, uout_hbm.at[pl.ds(base, b_per_w)])
        for cp in icps:
            cp.wait()
        pltpu.sync_copy(irows_v, iout_hbm.at[pl.ds(base, b_per_w)])

    return k


def kernel(users, items, user_table, item_table):
    B = users.shape[0]
    D = user_table.shape[1]
    k = _build(B, D, user_table.dtype)
    out = k(users.astype(jnp.int32), items.astype(jnp.int32),
            user_table, item_table)
    return (out[0], out[1])
